# baseline (speedup 1.0000x reference)
import functools

import jax
import jax.numpy as jnp
from jax import lax
from jax.experimental import pallas as pl
from jax.experimental.pallas import tpu as pltpu
from jax.experimental.pallas import tpu_sc as plsc

_BUCKET = 100000
_D = 64
_NC = 2
_NS = 16
_NW = _NC * _NS
_L = 16

_CROWS = 4
_CHUNK = _CROWS * 26
_NSLOT = 4

_AROWS = 125


def _make_aug_call():
    rows_per_w = _BUCKET // _NW
    n_chunks = rows_per_w // _AROWS
    mesh = plsc.VectorSubcoreMesh(core_axis_name="c", subcore_axis_name="s")

    scratch = (
        [pltpu.VMEM((_AROWS, _D), jnp.float32) for _ in range(6)]
        + [pltpu.SemaphoreType.DMA for _ in range(6)]
    )

    @functools.partial(
        pl.kernel,
        mesh=mesh,
        compiler_params=pltpu.CompilerParams(use_tc_tiling_on_sc=False),
        out_type=jax.ShapeDtypeStruct((2 * _BUCKET, _D), jnp.float32),
        scratch_types=scratch,
    )
    def aug_call(w_hbm, aug_hbm, *bufs):
        wv = bufs[0:2]
        pv = bufs[2:4]
        nv = bufs[4:6]
        in_s = bufs[6:8]
        po_s = bufs[8:10]
        ne_s = bufs[10:12]

        wid = lax.axis_index("s") * _NC + lax.axis_index("c")
        base = wid * rows_per_w

        def in_start(c, k):
            pltpu.async_copy(
                w_hbm.at[pl.ds(base + c * _AROWS, _AROWS)], wv[k], in_s[k])

        def in_wait(c, k):
            pltpu.make_async_copy(
                w_hbm.at[pl.ds(base + c * _AROWS, _AROWS)], wv[k],
                in_s[k]).wait()

        def out_start(c, k):
            pltpu.async_copy(
                pv[k], aug_hbm.at[pl.ds(_BUCKET + base + c * _AROWS,
                                        _AROWS)], po_s[k])
            pltpu.async_copy(
                nv[k], aug_hbm.at[pl.ds(base + c * _AROWS, _AROWS)], ne_s[k])

        def out_wait(c, k):
            pltpu.make_async_copy(
                pv[k], aug_hbm.at[pl.ds(_BUCKET + base + c * _AROWS,
                                        _AROWS)], po_s[k]).wait()
            pltpu.make_async_copy(
                nv[k], aug_hbm.at[pl.ds(base + c * _AROWS, _AROWS)],
                ne_s[k]).wait()

        in_start(0, 0)

        def chunk_body(c, carry):
            k = lax.rem(c, 2)
            for kk in range(2):
                @pl.when(k == kk)
                def _():
                    in_wait(c, kk)

                    @pl.when(c + 1 < n_chunks)
                    def _():
                        in_start(c + 1, 1 - kk)

                    @pl.when(c >= 2)
                    def _():
                        out_wait(c - 2, kk)

                    for r in range(_AROWS):
                        for d in range(_D // _L):
                            sl = pl.ds(d * _L, _L)
                            p = wv[kk][r, sl] * 0.5
                            pv[kk][r, sl] = p
                            nv[kk][r, sl] = -p
                    out_start(c, kk)
            return carry

        lax.fori_loop(0, n_chunks, chunk_body, 0)
        out_wait(n_chunks - 2, (n_chunks - 2) % 2)
        out_wait(n_chunks - 1, (n_chunks - 1) % 2)

    return aug_call


def _make_sc_call(batch, fields):
    rows_per_w = batch // _NW
    n_chunks = rows_per_w // _CROWS
    n_blocks = n_chunks // _NSLOT
    assert batch % _NW == 0 and rows_per_w % (_CROWS * _NSLOT) == 0
    splits = []
    o = 0
    while o < _CHUNK:
        w = min(128, _CHUNK - o)
        splits.append((o, w))
        o += w
    mesh = plsc.VectorSubcoreMesh(core_axis_name="c", subcore_axis_name="s")

    scratch = (
        [pltpu.VMEM((_L,), jnp.int32)]
        + [pltpu.VMEM((_CHUNK,), jnp.int32) for _ in range(_NSLOT)]
        + [pltpu.VMEM((_CHUNK,), jnp.int32) for _ in range(2 * _NSLOT)]
        + [pltpu.VMEM((_CHUNK, _D), jnp.float32) for _ in range(_NSLOT)]
        + [pltpu.SemaphoreType.DMA for _ in range(4 * _NSLOT)]
    )

    @functools.partial(
        pl.kernel,
        mesh=mesh,
        compiler_params=pltpu.CompilerParams(use_tc_tiling_on_sc=False),
        out_type=jax.ShapeDtypeStruct((batch, fields, _D), jnp.float32),
        scratch_types=scratch,
    )
    def sc_call(aug_hbm, ids_hbm, hp_hbm, out_hbm, hp_v, *bufs):
        ids_v = bufs[0:_NSLOT]
        idx0_v = bufs[_NSLOT:2 * _NSLOT]
        idx1_v = bufs[2 * _NSLOT:3 * _NSLOT]
        r_v = bufs[3 * _NSLOT:4 * _NSLOT]
        ids_s = bufs[4 * _NSLOT:5 * _NSLOT]
        g0_s = bufs[5 * _NSLOT:6 * _NSLOT]
        ga_s = bufs[6 * _NSLOT:7 * _NSLOT]
        out_s = bufs[7 * _NSLOT:8 * _NSLOT]

        wid = lax.axis_index("s") * _NC + lax.axis_index("c")
        base = wid * rows_per_w * fields
        brow = wid * rows_per_w

        pltpu.sync_copy(hp_hbm, hp_v)
        hpv = hp_v[...]
        ha0, ha1 = hpv[0], hpv[1]
        hb0, hb1 = hpv[2], hpv[3]
        sa0, sa1 = hpv[4], hpv[5]
        sb0, sb1 = hpv[6], hpv[7]

        def ids_start(c, k):
            pltpu.async_copy(
                ids_hbm.at[pl.ds(base + c * _CHUNK, _CHUNK)], ids_v[k],
                ids_s[k])

        def gather0_start(k):
            for (o, w) in splits:
                pltpu.async_copy(aug_hbm.at[idx0_v[k].at[pl.ds(o, w)]],
                                 r_v[k].at[pl.ds(o, w)], g0_s[k])

        def gadd_start(k):
            for (o, w) in splits:
                pltpu.async_copy(aug_hbm.at[idx1_v[k].at[pl.ds(o, w)]],
                                 r_v[k].at[pl.ds(o, w)], ga_s[k],
                                 add=True)

        def out_start(c, k):
            for r in range(_CROWS):
                pltpu.async_copy(
                    r_v[k].at[pl.ds(r * fields, fields)],
                    out_hbm.at[brow + c * _CROWS + r],
                    out_s[k])

        def gather0_wait(k):
            for (o, w) in splits:
                pltpu.make_async_copy(aug_hbm.at[idx0_v[k].at[pl.ds(o, w)]],
                                      r_v[k].at[pl.ds(o, w)], g0_s[k]).wait()

        def gadd_wait(k):
            for (o, w) in splits:
                pltpu.make_async_copy(aug_hbm.at[idx1_v[k].at[pl.ds(o, w)]],
                                      r_v[k].at[pl.ds(o, w)], ga_s[k]).wait()

        def out_wait(c, k):
            for r in range(_CROWS):
                pltpu.make_async_copy(
                    r_v[k].at[pl.ds(r * fields, fields)],
                    out_hbm.at[brow + c * _CROWS + r],
                    out_s[k]).wait()

        offs = list(range(0, _CHUNK - _L + 1, _L))
        if _CHUNK % _L:
            offs.append(_CHUNK - _L)

        def compute_idx(k):
            for o in offs:
                v = ids_v[k][pl.ds(o, _L)]
                b0 = jnp.mod(v * ha0 + hb0, _BUCKET)
                m0 = (v * sa0 + sb0) & 1
                idx0_v[k][pl.ds(o, _L)] = b0 + m0 * _BUCKET
                b1 = jnp.mod(v * ha1 + hb1, _BUCKET)
                m1 = (v * sa1 + sb1) & 1
                idx1_v[k][pl.ds(o, _L)] = b1 + m1 * _BUCKET

        for k in range(_NSLOT):
            ids_start(k, k)

        def block_body(b, carry):
            for k in range(_NSLOT):
                c = b * _NSLOT + k
                pltpu.make_async_copy(
                    ids_hbm.at[pl.ds(base + c * _CHUNK, _CHUNK)], ids_v[k],
                    ids_s[k]).wait()
                compute_idx(k)

                @pl.when(b < n_blocks - 1)
                def _():
                    ids_start(c + _NSLOT, k)

                @pl.when(b >= 1)
                def _():
                    out_wait(c - _NSLOT, k)

                gather0_start(k)
                k1 = (k - 1) % _NSLOT
                if k == 0:
                    @pl.when(b >= 1)
                    def _():
                        gather0_wait(k1)
                        gadd_start(k1)
                else:
                    gather0_wait(k1)
                    gadd_start(k1)
                k2 = (k - 2) % _NSLOT
                c2 = c - 2
                if k in (0, 1):
                    @pl.when(b >= 1)
                    def _():
                        gadd_wait(k2)
                        out_start(c2, k2)
                else:
                    gadd_wait(k2)
                    out_start(c2, k2)
            return carry

        lax.fori_loop(0, n_blocks, block_body, 0)

        n = n_chunks
        gather0_wait(_NSLOT - 1)
        gadd_start(_NSLOT - 1)
        gadd_wait(_NSLOT - 2)
        out_start(n - 2, _NSLOT - 2)
        gadd_wait(_NSLOT - 1)
        out_start(n - 1, _NSLOT - 1)
        for k in range(_NSLOT):
            out_wait(n - _NSLOT + k, k)

    return sc_call


def kernel(input_ids, weight, hash_a, hash_b, sign_a, sign_b):
    batch, fields = input_ids.shape
    aug = _make_aug_call()(weight)
    ids_flat = input_ids.reshape(batch * fields)
    hp = jnp.concatenate(
        [hash_a, hash_b, sign_a, sign_b,
         jnp.zeros((_L - 8,), jnp.int32)]).astype(jnp.int32)
    return _make_sc_call(batch, fields)(aug, ids_flat, hp)


_make_sc_call = functools.lru_cache(None)(_make_sc_call)
_make_aug_call = functools.lru_cache(None)(_make_aug_call)


# speedup vs baseline: 1.2455x; 1.2455x over previous
import functools

import jax
import jax.numpy as jnp
from jax import lax
from jax.experimental import pallas as pl
from jax.experimental.pallas import tpu as pltpu
from jax.experimental.pallas import tpu_sc as plsc

_BUCKET = 100000
_D = 64
_NC = 2
_NS = 16
_NW = _NC * _NS
_L = 16

_CHUNK = 256
_NSLOT = 4

_AROWS = 125


def _make_aug_call():
    rows_per_w = _BUCKET // _NW
    n_chunks = rows_per_w // _AROWS
    mesh = plsc.VectorSubcoreMesh(core_axis_name="c", subcore_axis_name="s")

    scratch = (
        [pltpu.VMEM((_AROWS, _D), jnp.float32) for _ in range(6)]
        + [pltpu.SemaphoreType.DMA for _ in range(6)]
    )

    @functools.partial(
        pl.kernel,
        mesh=mesh,
        compiler_params=pltpu.CompilerParams(use_tc_tiling_on_sc=False),
        out_type=jax.ShapeDtypeStruct((2 * _BUCKET, _D), jnp.float32),
        scratch_types=scratch,
    )
    def aug_call(w_hbm, aug_hbm, *bufs):
        wv = bufs[0:2]
        pv = bufs[2:4]
        nv = bufs[4:6]
        in_s = bufs[6:8]
        po_s = bufs[8:10]
        ne_s = bufs[10:12]

        wid = lax.axis_index("s") * _NC + lax.axis_index("c")
        base = wid * rows_per_w

        def in_start(c, k):
            pltpu.async_copy(
                w_hbm.at[pl.ds(base + c * _AROWS, _AROWS)], wv[k], in_s[k])

        def in_wait(c, k):
            pltpu.make_async_copy(
                w_hbm.at[pl.ds(base + c * _AROWS, _AROWS)], wv[k],
                in_s[k]).wait()

        def out_start(c, k):
            pltpu.async_copy(
                pv[k], aug_hbm.at[pl.ds(_BUCKET + base + c * _AROWS,
                                        _AROWS)], po_s[k])
            pltpu.async_copy(
                nv[k], aug_hbm.at[pl.ds(base + c * _AROWS, _AROWS)], ne_s[k])

        def out_wait(c, k):
            pltpu.make_async_copy(
                pv[k], aug_hbm.at[pl.ds(_BUCKET + base + c * _AROWS,
                                        _AROWS)], po_s[k]).wait()
            pltpu.make_async_copy(
                nv[k], aug_hbm.at[pl.ds(base + c * _AROWS, _AROWS)],
                ne_s[k]).wait()

        in_start(0, 0)

        def chunk_body(c, carry):
            k = lax.rem(c, 2)
            for kk in range(2):
                @pl.when(k == kk)
                def _():
                    in_wait(c, kk)

                    @pl.when(c + 1 < n_chunks)
                    def _():
                        in_start(c + 1, 1 - kk)

                    @pl.when(c >= 2)
                    def _():
                        out_wait(c - 2, kk)

                    for r in range(_AROWS):
                        for d in range(_D // _L):
                            sl = pl.ds(d * _L, _L)
                            p = wv[kk][r, sl] * 0.5
                            pv[kk][r, sl] = p
                            nv[kk][r, sl] = -p
                    out_start(c, kk)
            return carry

        lax.fori_loop(0, n_chunks, chunk_body, 0)
        out_wait(n_chunks - 2, (n_chunks - 2) % 2)
        out_wait(n_chunks - 1, (n_chunks - 1) % 2)

    return aug_call


def _make_sc_call(n_total):
    n_per_w = n_total // _NW
    n_chunks = n_per_w // _CHUNK
    n_blocks = n_chunks // _NSLOT
    assert n_total % (_NW * _CHUNK * _NSLOT) == 0
    assert _CHUNK % 128 == 0
    n_split = _CHUNK // 128
    mesh = plsc.VectorSubcoreMesh(core_axis_name="c", subcore_axis_name="s")

    scratch = (
        [pltpu.VMEM((_L,), jnp.int32)]
        + [pltpu.VMEM((_CHUNK,), jnp.int32) for _ in range(_NSLOT)]
        + [pltpu.VMEM((n_split, 128), jnp.int32)
           for _ in range(2 * _NSLOT)]
        + [pltpu.VMEM((_CHUNK, _D), jnp.float32) for _ in range(_NSLOT)]
        + [pltpu.SemaphoreType.DMA for _ in range(4 * _NSLOT)]
    )

    @functools.partial(
        pl.kernel,
        mesh=mesh,
        compiler_params=pltpu.CompilerParams(use_tc_tiling_on_sc=False),
        out_type=jax.ShapeDtypeStruct((n_total, _D), jnp.float32),
        scratch_types=scratch,
    )
    def sc_call(aug_hbm, ids_hbm, hp_hbm, out_hbm, hp_v, *bufs):
        ids_v = bufs[0:_NSLOT]
        idx0_v = bufs[_NSLOT:2 * _NSLOT]
        idx1_v = bufs[2 * _NSLOT:3 * _NSLOT]
        r_v = bufs[3 * _NSLOT:4 * _NSLOT]
        ids_s = bufs[4 * _NSLOT:5 * _NSLOT]
        g0_s = bufs[5 * _NSLOT:6 * _NSLOT]
        ga_s = bufs[6 * _NSLOT:7 * _NSLOT]
        out_s = bufs[7 * _NSLOT:8 * _NSLOT]

        wid = lax.axis_index("s") * _NC + lax.axis_index("c")
        base = wid * n_per_w

        pltpu.sync_copy(hp_hbm, hp_v)
        hpv = hp_v[...]
        ha0, ha1 = hpv[0], hpv[1]
        hb0, hb1 = hpv[2], hpv[3]
        sa0, sa1 = hpv[4], hpv[5]
        sb0, sb1 = hpv[6], hpv[7]

        def ids_start(c, k):
            pltpu.async_copy(
                ids_hbm.at[pl.ds(base + c * _CHUNK, _CHUNK)], ids_v[k],
                ids_s[k])

        def gather0_start(k):
            for j in range(n_split):
                pltpu.async_copy(aug_hbm.at[idx0_v[k].at[j]],
                                 r_v[k].at[pl.ds(j * 128, 128)], g0_s[k])

        def gadd_start(k):
            for j in range(n_split):
                pltpu.async_copy(aug_hbm.at[idx1_v[k].at[j]],
                                 r_v[k].at[pl.ds(j * 128, 128)], ga_s[k],
                                 add=True)

        def out_start(c, k):
            pltpu.async_copy(
                r_v[k], out_hbm.at[pl.ds(base + c * _CHUNK, _CHUNK)],
                out_s[k])

        def gather0_wait(k):
            for j in range(n_split):
                pltpu.make_async_copy(aug_hbm.at[idx0_v[k].at[j]],
                                      r_v[k].at[pl.ds(j * 128, 128)],
                                      g0_s[k]).wait()

        def gadd_wait(k):
            for j in range(n_split):
                pltpu.make_async_copy(aug_hbm.at[idx1_v[k].at[j]],
                                      r_v[k].at[pl.ds(j * 128, 128)],
                                      ga_s[k]).wait()

        def out_wait(c, k):
            pltpu.make_async_copy(
                r_v[k], out_hbm.at[pl.ds(base + c * _CHUNK, _CHUNK)],
                out_s[k]).wait()

        def compute_idx(k):
            def jbody(j, carry):
                for gg in range(128 // _L):
                    o = gg * _L
                    v = ids_v[k][pl.ds(j * 128 + o, _L)]
                    b0 = jnp.mod(v * ha0 + hb0, _BUCKET)
                    m0 = (v * sa0 + sb0) & 1
                    idx0_v[k][j, pl.ds(o, _L)] = b0 + m0 * _BUCKET
                    b1 = jnp.mod(v * ha1 + hb1, _BUCKET)
                    m1 = (v * sa1 + sb1) & 1
                    idx1_v[k][j, pl.ds(o, _L)] = b1 + m1 * _BUCKET
                return carry

            lax.fori_loop(0, n_split, jbody, 0)

        for k in range(_NSLOT):
            ids_start(k, k)

        def block_body(b, carry):
            for k in range(_NSLOT):
                c = b * _NSLOT + k
                pltpu.make_async_copy(
                    ids_hbm.at[pl.ds(base + c * _CHUNK, _CHUNK)], ids_v[k],
                    ids_s[k]).wait()
                compute_idx(k)

                @pl.when(b < n_blocks - 1)
                def _():
                    ids_start(c + _NSLOT, k)

                @pl.when(b >= 1)
                def _():
                    out_wait(c - _NSLOT, k)

                gather0_start(k)
                k1 = (k - 1) % _NSLOT
                if k == 0:
                    @pl.when(b >= 1)
                    def _():
                        gather0_wait(k1)
                        gadd_start(k1)
                else:
                    gather0_wait(k1)
                    gadd_start(k1)
                k2 = (k - 2) % _NSLOT
                c2 = c - 2
                if k in (0, 1):
                    @pl.when(b >= 1)
                    def _():
                        gadd_wait(k2)
                        out_start(c2, k2)
                else:
                    gadd_wait(k2)
                    out_start(c2, k2)
            return carry

        lax.fori_loop(0, n_blocks, block_body, 0)

        n = n_chunks
        gather0_wait(_NSLOT - 1)
        gadd_start(_NSLOT - 1)
        gadd_wait(_NSLOT - 2)
        out_start(n - 2, _NSLOT - 2)
        gadd_wait(_NSLOT - 1)
        out_start(n - 1, _NSLOT - 1)
        for k in range(_NSLOT):
            out_wait(n - _NSLOT + k, k)

    return sc_call


def kernel(input_ids, weight, hash_a, hash_b, sign_a, sign_b):
    batch, fields = input_ids.shape
    n_total = batch * fields
    aug = _make_aug_call()(weight)
    ids_flat = input_ids.reshape(n_total)
    hp = jnp.concatenate(
        [hash_a, hash_b, sign_a, sign_b,
         jnp.zeros((_L - 8,), jnp.int32)]).astype(jnp.int32)
    out = _make_sc_call(n_total)(aug, ids_flat, hp)
    return out.reshape(batch, fields, _D)


_make_sc_call = functools.lru_cache(None)(_make_sc_call)
_make_aug_call = functools.lru_cache(None)(_make_aug_call)


# speedup vs baseline: 1.2715x; 1.0209x over previous
import functools

import jax
import jax.numpy as jnp
from jax import lax
from jax.experimental import pallas as pl
from jax.experimental.pallas import tpu as pltpu
from jax.experimental.pallas import tpu_sc as plsc

_BUCKET = 100000
_D = 64
_NC = 2
_NS = 16
_NW = _NC * _NS
_L = 16

_IW = 104
_CHUNK = 416
_NSLOT = 4

_AROWS = 125


def _make_aug_call():
    rows_per_w = _BUCKET // _NW
    n_chunks = rows_per_w // _AROWS
    mesh = plsc.VectorSubcoreMesh(core_axis_name="c", subcore_axis_name="s")

    scratch = (
        [pltpu.VMEM((_AROWS, _D), jnp.float32) for _ in range(6)]
        + [pltpu.SemaphoreType.DMA for _ in range(6)]
    )

    @functools.partial(
        pl.kernel,
        mesh=mesh,
        compiler_params=pltpu.CompilerParams(use_tc_tiling_on_sc=False),
        out_type=jax.ShapeDtypeStruct((2 * _BUCKET, _D), jnp.float32),
        scratch_types=scratch,
    )
    def aug_call(w_hbm, aug_hbm, *bufs):
        wv = bufs[0:2]
        pv = bufs[2:4]
        nv = bufs[4:6]
        in_s = bufs[6:8]
        po_s = bufs[8:10]
        ne_s = bufs[10:12]

        wid = lax.axis_index("s") * _NC + lax.axis_index("c")
        base = wid * rows_per_w

        def in_start(c, k):
            pltpu.async_copy(
                w_hbm.at[pl.ds(base + c * _AROWS, _AROWS)], wv[k], in_s[k])

        def in_wait(c, k):
            pltpu.make_async_copy(
                w_hbm.at[pl.ds(base + c * _AROWS, _AROWS)], wv[k],
                in_s[k]).wait()

        def out_start(c, k):
            pltpu.async_copy(
                pv[k], aug_hbm.at[pl.ds(_BUCKET + base + c * _AROWS,
                                        _AROWS)], po_s[k])
            pltpu.async_copy(
                nv[k], aug_hbm.at[pl.ds(base + c * _AROWS, _AROWS)], ne_s[k])

        def out_wait(c, k):
            pltpu.make_async_copy(
                pv[k], aug_hbm.at[pl.ds(_BUCKET + base + c * _AROWS,
                                        _AROWS)], po_s[k]).wait()
            pltpu.make_async_copy(
                nv[k], aug_hbm.at[pl.ds(base + c * _AROWS, _AROWS)],
                ne_s[k]).wait()

        in_start(0, 0)

        def chunk_body(c, carry):
            k = lax.rem(c, 2)
            for kk in range(2):
                @pl.when(k == kk)
                def _():
                    in_wait(c, kk)

                    @pl.when(c + 1 < n_chunks)
                    def _():
                        in_start(c + 1, 1 - kk)

                    @pl.when(c >= 2)
                    def _():
                        out_wait(c - 2, kk)

                    for r in range(_AROWS):
                        for d in range(_D // _L):
                            sl = pl.ds(d * _L, _L)
                            p = wv[kk][r, sl] * 0.5
                            pv[kk][r, sl] = p
                            nv[kk][r, sl] = -p
                    out_start(c, kk)
            return carry

        lax.fori_loop(0, n_chunks, chunk_body, 0)
        out_wait(n_chunks - 2, (n_chunks - 2) % 2)
        out_wait(n_chunks - 1, (n_chunks - 1) % 2)

    return aug_call


def _make_sc_call(n_total):
    n_per_w = n_total // _NW
    n_chunks = n_per_w // _CHUNK
    n_blocks = n_chunks // _NSLOT
    assert n_total % (_NW * _CHUNK * _NSLOT) == 0
    assert _CHUNK % _IW == 0
    n_split = _CHUNK // _IW
    mesh = plsc.VectorSubcoreMesh(core_axis_name="c", subcore_axis_name="s")

    scratch = (
        [pltpu.VMEM((_L,), jnp.int32)]
        + [pltpu.VMEM((_CHUNK,), jnp.int32) for _ in range(_NSLOT)]
        + [pltpu.VMEM((n_split, _IW), jnp.int32)
           for _ in range(2 * _NSLOT)]
        + [pltpu.VMEM((_CHUNK, _D), jnp.float32) for _ in range(_NSLOT)]
        + [pltpu.SemaphoreType.DMA for _ in range(4 * _NSLOT)]
    )

    @functools.partial(
        pl.kernel,
        mesh=mesh,
        compiler_params=pltpu.CompilerParams(use_tc_tiling_on_sc=False),
        out_type=jax.ShapeDtypeStruct((n_total, _D), jnp.float32),
        scratch_types=scratch,
    )
    def sc_call(aug_hbm, ids_hbm, hp_hbm, out_hbm, hp_v, *bufs):
        ids_v = bufs[0:_NSLOT]
        idx0_v = bufs[_NSLOT:2 * _NSLOT]
        idx1_v = bufs[2 * _NSLOT:3 * _NSLOT]
        r_v = bufs[3 * _NSLOT:4 * _NSLOT]
        ids_s = bufs[4 * _NSLOT:5 * _NSLOT]
        g0_s = bufs[5 * _NSLOT:6 * _NSLOT]
        ga_s = bufs[6 * _NSLOT:7 * _NSLOT]
        out_s = bufs[7 * _NSLOT:8 * _NSLOT]

        wid = lax.axis_index("s") * _NC + lax.axis_index("c")
        base = wid * n_per_w

        pltpu.sync_copy(hp_hbm, hp_v)
        hpv = hp_v[...]
        ha0, ha1 = hpv[0], hpv[1]
        hb0, hb1 = hpv[2], hpv[3]
        sa0, sa1 = hpv[4], hpv[5]
        sb0, sb1 = hpv[6], hpv[7]

        def ids_start(c, k):
            pltpu.async_copy(
                ids_hbm.at[pl.ds(base + c * _CHUNK, _CHUNK)], ids_v[k],
                ids_s[k])

        def gather0_start(k):
            for j in range(n_split):
                pltpu.async_copy(aug_hbm.at[idx0_v[k].at[j]],
                                 r_v[k].at[pl.ds(j * _IW, _IW)], g0_s[k])

        def gadd_start(k):
            for j in range(n_split):
                pltpu.async_copy(aug_hbm.at[idx1_v[k].at[j]],
                                 r_v[k].at[pl.ds(j * _IW, _IW)], ga_s[k],
                                 add=True)

        def out_start(c, k):
            pltpu.async_copy(
                r_v[k], out_hbm.at[pl.ds(base + c * _CHUNK, _CHUNK)],
                out_s[k])

        def gather0_wait(k):
            for j in range(n_split):
                pltpu.make_async_copy(aug_hbm.at[idx0_v[k].at[j]],
                                      r_v[k].at[pl.ds(j * _IW, _IW)],
                                      g0_s[k]).wait()

        def gadd_wait(k):
            for j in range(n_split):
                pltpu.make_async_copy(aug_hbm.at[idx1_v[k].at[j]],
                                      r_v[k].at[pl.ds(j * _IW, _IW)],
                                      ga_s[k]).wait()

        def out_wait(c, k):
            pltpu.make_async_copy(
                r_v[k], out_hbm.at[pl.ds(base + c * _CHUNK, _CHUNK)],
                out_s[k]).wait()

        row_offs = list(range(0, _IW - _L + 1, _L))
        if _IW % _L:
            row_offs.append(_IW - _L)

        def compute_idx(k):
            def jbody(j, carry):
                for o in row_offs:
                    v = ids_v[k][pl.ds(j * _IW + o, _L)]
                    b0 = jnp.mod(v * ha0 + hb0, _BUCKET)
                    m0 = (v * sa0 + sb0) & 1
                    idx0_v[k][j, pl.ds(o, _L)] = b0 + m0 * _BUCKET
                    b1 = jnp.mod(v * ha1 + hb1, _BUCKET)
                    m1 = (v * sa1 + sb1) & 1
                    idx1_v[k][j, pl.ds(o, _L)] = b1 + m1 * _BUCKET
                return carry

            lax.fori_loop(0, n_split, jbody, 0)

        for k in range(_NSLOT):
            ids_start(k, k)

        def block_body(b, carry):
            for k in range(_NSLOT):
                c = b * _NSLOT + k
                pltpu.make_async_copy(
                    ids_hbm.at[pl.ds(base + c * _CHUNK, _CHUNK)], ids_v[k],
                    ids_s[k]).wait()
                compute_idx(k)

                @pl.when(b < n_blocks - 1)
                def _():
                    ids_start(c + _NSLOT, k)

                @pl.when(b >= 1)
                def _():
                    out_wait(c - _NSLOT, k)

                gather0_start(k)
                k1 = (k - 1) % _NSLOT
                if k == 0:
                    @pl.when(b >= 1)
                    def _():
                        gather0_wait(k1)
                        gadd_start(k1)
                else:
                    gather0_wait(k1)
                    gadd_start(k1)
                k2 = (k - 2) % _NSLOT
                c2 = c - 2
                if k in (0, 1):
                    @pl.when(b >= 1)
                    def _():
                        gadd_wait(k2)
                        out_start(c2, k2)
                else:
                    gadd_wait(k2)
                    out_start(c2, k2)
            return carry

        lax.fori_loop(0, n_blocks, block_body, 0)

        n = n_chunks
        gather0_wait(_NSLOT - 1)
        gadd_start(_NSLOT - 1)
        gadd_wait(_NSLOT - 2)
        out_start(n - 2, _NSLOT - 2)
        gadd_wait(_NSLOT - 1)
        out_start(n - 1, _NSLOT - 1)
        for k in range(_NSLOT):
            out_wait(n - _NSLOT + k, k)

    return sc_call


def kernel(input_ids, weight, hash_a, hash_b, sign_a, sign_b):
    batch, fields = input_ids.shape
    n_total = batch * fields
    aug = _make_aug_call()(weight)
    ids_flat = input_ids.reshape(n_total)
    hp = jnp.concatenate(
        [hash_a, hash_b, sign_a, sign_b,
         jnp.zeros((_L - 8,), jnp.int32)]).astype(jnp.int32)
    out = _make_sc_call(n_total)(aug, ids_flat, hp)
    return out.reshape(batch, fields, _D)


_make_sc_call = functools.lru_cache(None)(_make_sc_call)
_make_aug_call = functools.lru_cache(None)(_make_aug_call)


# speedup vs baseline: 1.2765x; 1.0039x over previous
import functools

import jax
import jax.numpy as jnp
from jax import lax
from jax.experimental import pallas as pl
from jax.experimental.pallas import tpu as pltpu
from jax.experimental.pallas import tpu_sc as plsc

_BUCKET = 100000
_D = 64
_NC = 2
_NS = 16
_NW = _NC * _NS
_L = 16

_IW = 104
_CHUNK = 416
_NSLOT = 4

_AROWS = 125


def _make_aug_call():
    rows_per_w = _BUCKET // _NW
    n_chunks = rows_per_w // _AROWS
    mesh = plsc.VectorSubcoreMesh(core_axis_name="c", subcore_axis_name="s")

    scratch = (
        [pltpu.VMEM((_AROWS, _D), jnp.float32) for _ in range(6)]
        + [pltpu.SemaphoreType.DMA for _ in range(6)]
    )

    @functools.partial(
        pl.kernel,
        mesh=mesh,
        compiler_params=pltpu.CompilerParams(use_tc_tiling_on_sc=False),
        out_type=jax.ShapeDtypeStruct((2 * _BUCKET, _D), jnp.float32),
        scratch_types=scratch,
    )
    def aug_call(w_hbm, aug_hbm, *bufs):
        wv = bufs[0:2]
        pv = bufs[2:4]
        nv = bufs[4:6]
        in_s = bufs[6:8]
        po_s = bufs[8:10]
        ne_s = bufs[10:12]

        wid = lax.axis_index("s") * _NC + lax.axis_index("c")
        base = wid * rows_per_w

        def in_start(c, k):
            pltpu.async_copy(
                w_hbm.at[pl.ds(base + c * _AROWS, _AROWS)], wv[k], in_s[k])

        def in_wait(c, k):
            pltpu.make_async_copy(
                w_hbm.at[pl.ds(base + c * _AROWS, _AROWS)], wv[k],
                in_s[k]).wait()

        def out_start(c, k):
            pltpu.async_copy(
                pv[k], aug_hbm.at[pl.ds(_BUCKET + base + c * _AROWS,
                                        _AROWS)], po_s[k])
            pltpu.async_copy(
                nv[k], aug_hbm.at[pl.ds(base + c * _AROWS, _AROWS)], ne_s[k])

        def out_wait(c, k):
            pltpu.make_async_copy(
                pv[k], aug_hbm.at[pl.ds(_BUCKET + base + c * _AROWS,
                                        _AROWS)], po_s[k]).wait()
            pltpu.make_async_copy(
                nv[k], aug_hbm.at[pl.ds(base + c * _AROWS, _AROWS)],
                ne_s[k]).wait()

        in_start(0, 0)

        def chunk_body(c, carry):
            k = lax.rem(c, 2)
            for kk in range(2):
                @pl.when(k == kk)
                def _():
                    in_wait(c, kk)

                    @pl.when(c + 1 < n_chunks)
                    def _():
                        in_start(c + 1, 1 - kk)

                    @pl.when(c >= 2)
                    def _():
                        out_wait(c - 2, kk)

                    for r in range(_AROWS):
                        for d in range(_D // _L):
                            sl = pl.ds(d * _L, _L)
                            p = wv[kk][r, sl] * 0.5
                            pv[kk][r, sl] = p
                            nv[kk][r, sl] = -p
                    out_start(c, kk)
            return carry

        lax.fori_loop(0, n_chunks, chunk_body, 0)
        out_wait(n_chunks - 2, (n_chunks - 2) % 2)
        out_wait(n_chunks - 1, (n_chunks - 1) % 2)

    return aug_call


def _make_sc_call(n_total):
    n_per_w = n_total // _NW
    n_chunks = n_per_w // _CHUNK
    n_blocks = n_chunks // _NSLOT
    assert n_total % (_NW * _CHUNK * _NSLOT) == 0
    assert _CHUNK % _IW == 0
    n_split = _CHUNK // _IW
    mesh = plsc.VectorSubcoreMesh(core_axis_name="c", subcore_axis_name="s")

    scratch = (
        [pltpu.VMEM((_L,), jnp.int32)]
        + [pltpu.VMEM((_CHUNK // 26, 26), jnp.int32)
           for _ in range(_NSLOT)]
        + [pltpu.VMEM((n_split, _IW), jnp.int32)
           for _ in range(2 * _NSLOT)]
        + [pltpu.VMEM((_CHUNK, _D), jnp.float32) for _ in range(_NSLOT)]
        + [pltpu.SemaphoreType.DMA for _ in range(4 * _NSLOT)]
    )

    @functools.partial(
        pl.kernel,
        mesh=mesh,
        compiler_params=pltpu.CompilerParams(use_tc_tiling_on_sc=False),
        out_type=jax.ShapeDtypeStruct((n_total, _D), jnp.float32),
        scratch_types=scratch,
    )
    def sc_call(aug_hbm, ids_hbm, hp_hbm, out_hbm, hp_v, *bufs):
        ids_v = bufs[0:_NSLOT]
        idx0_v = bufs[_NSLOT:2 * _NSLOT]
        idx1_v = bufs[2 * _NSLOT:3 * _NSLOT]
        r_v = bufs[3 * _NSLOT:4 * _NSLOT]
        ids_s = bufs[4 * _NSLOT:5 * _NSLOT]
        g0_s = bufs[5 * _NSLOT:6 * _NSLOT]
        ga_s = bufs[6 * _NSLOT:7 * _NSLOT]
        out_s = bufs[7 * _NSLOT:8 * _NSLOT]

        wid = lax.axis_index("s") * _NC + lax.axis_index("c")
        base = wid * n_per_w

        pltpu.sync_copy(hp_hbm, hp_v)
        hpv = hp_v[...]
        ha0, ha1 = hpv[0], hpv[1]
        hb0, hb1 = hpv[2], hpv[3]
        sa0, sa1 = hpv[4], hpv[5]
        sb0, sb1 = hpv[6], hpv[7]

        crows = _CHUNK // 26
        brow = wid * (n_per_w // 26)

        def ids_start(c, k):
            pltpu.async_copy(
                ids_hbm.at[pl.ds(brow + c * crows, crows)], ids_v[k],
                ids_s[k])

        def gather0_start(k):
            for j in range(n_split):
                pltpu.async_copy(aug_hbm.at[idx0_v[k].at[j]],
                                 r_v[k].at[pl.ds(j * _IW, _IW)], g0_s[k])

        def gadd_start(k):
            for j in range(n_split):
                pltpu.async_copy(aug_hbm.at[idx1_v[k].at[j]],
                                 r_v[k].at[pl.ds(j * _IW, _IW)], ga_s[k],
                                 add=True)

        def out_start(c, k):
            pltpu.async_copy(
                r_v[k], out_hbm.at[pl.ds(base + c * _CHUNK, _CHUNK)],
                out_s[k])

        def gather0_wait(k):
            for j in range(n_split):
                pltpu.make_async_copy(aug_hbm.at[idx0_v[k].at[j]],
                                      r_v[k].at[pl.ds(j * _IW, _IW)],
                                      g0_s[k]).wait()

        def gadd_wait(k):
            for j in range(n_split):
                pltpu.make_async_copy(aug_hbm.at[idx1_v[k].at[j]],
                                      r_v[k].at[pl.ds(j * _IW, _IW)],
                                      ga_s[k]).wait()

        def out_wait(c, k):
            pltpu.make_async_copy(
                r_v[k], out_hbm.at[pl.ds(base + c * _CHUNK, _CHUNK)],
                out_s[k]).wait()

        rpj = _IW // 26

        def compute_idx(k):
            def jbody(j, carry):
                for rr in range(rpj):
                    for o2 in (0, 26 - _L):
                        v = ids_v[k][j * rpj + rr, pl.ds(o2, _L)]
                        o = rr * 26 + o2
                        b0 = jnp.mod(v * ha0 + hb0, _BUCKET)
                        m0 = (v * sa0 + sb0) & 1
                        idx0_v[k][j, pl.ds(o, _L)] = b0 + m0 * _BUCKET
                        b1 = jnp.mod(v * ha1 + hb1, _BUCKET)
                        m1 = (v * sa1 + sb1) & 1
                        idx1_v[k][j, pl.ds(o, _L)] = b1 + m1 * _BUCKET
                return carry

            lax.fori_loop(0, n_split, jbody, 0)

        for k in range(_NSLOT):
            ids_start(k, k)

        def block_body(b, carry):
            for k in range(_NSLOT):
                c = b * _NSLOT + k
                pltpu.make_async_copy(
                    ids_hbm.at[pl.ds(brow + c * crows, crows)], ids_v[k],
                    ids_s[k]).wait()
                compute_idx(k)

                @pl.when(b < n_blocks - 1)
                def _():
                    ids_start(c + _NSLOT, k)

                @pl.when(b >= 1)
                def _():
                    out_wait(c - _NSLOT, k)

                gather0_start(k)
                k1 = (k - 1) % _NSLOT
                if k == 0:
                    @pl.when(b >= 1)
                    def _():
                        gather0_wait(k1)
                        gadd_start(k1)
                else:
                    gather0_wait(k1)
                    gadd_start(k1)
                k2 = (k - 2) % _NSLOT
                c2 = c - 2
                if k in (0, 1):
                    @pl.when(b >= 1)
                    def _():
                        gadd_wait(k2)
                        out_start(c2, k2)
                else:
                    gadd_wait(k2)
                    out_start(c2, k2)
            return carry

        lax.fori_loop(0, n_blocks, block_body, 0)

        n = n_chunks
        gather0_wait(_NSLOT - 1)
        gadd_start(_NSLOT - 1)
        gadd_wait(_NSLOT - 2)
        out_start(n - 2, _NSLOT - 2)
        gadd_wait(_NSLOT - 1)
        out_start(n - 1, _NSLOT - 1)
        for k in range(_NSLOT):
            out_wait(n - _NSLOT + k, k)

    return sc_call


def kernel(input_ids, weight, hash_a, hash_b, sign_a, sign_b):
    batch, fields = input_ids.shape
    n_total = batch * fields
    aug = _make_aug_call()(weight)
    hp = jnp.concatenate(
        [hash_a, hash_b, sign_a, sign_b,
         jnp.zeros((_L - 8,), jnp.int32)]).astype(jnp.int32)
    out = _make_sc_call(n_total)(aug, input_ids, hp)
    return out.reshape(batch, fields, _D)


_make_sc_call = functools.lru_cache(None)(_make_sc_call)
_make_aug_call = functools.lru_cache(None)(_make_aug_call)
